# baseline (speedup 1.0000x reference)
import functools

import jax
import jax.numpy as jnp
from jax import lax
from jax.experimental import pallas as pl
from jax.experimental.pallas import tpu as pltpu
from jax.experimental.pallas import tpu_sc as plsc

NC = 2
NS = 16
NW = NC * NS
L = 16
CH = 16


@functools.lru_cache(maxsize=None)
def _build(B, K):
    assert B % (8 * NW) == 0
    b_per_w = B // NW
    n_chunks = b_per_w // CH
    n_k = K // L
    mesh = plsc.VectorSubcoreMesh(core_axis_name="c", subcore_axis_name="s")

    @functools.partial(
        pl.kernel,
        out_type=jax.ShapeDtypeStruct((B,), jnp.float32),
        mesh=mesh,
        compiler_params=pltpu.CompilerParams(needs_layout_passes=False),
        scratch_types=[
            pltpu.VMEM((b_per_w, 3), jnp.int32),
            pltpu.VMEM((n_chunks, CH), jnp.int32),
            pltpu.VMEM((n_chunks, CH), jnp.int32),
            pltpu.VMEM((n_chunks, CH), jnp.int32),
            pltpu.VMEM((n_chunks, CH), jnp.int32),
            pltpu.VMEM((n_chunks, CH), jnp.int32),
            pltpu.VMEM((CH, K), jnp.float32),
            pltpu.VMEM((CH, K), jnp.float32),
            pltpu.VMEM((CH, K), jnp.float32),
            pltpu.VMEM((b_per_w,), jnp.float32),
            pltpu.VMEM((L, L), jnp.float32),
            pltpu.SemaphoreType.DMA,
        ],
    )
    def parafac(idx_h, f0, f1, f2, out,
                idxbuf, blk0, blk1, blk2, sub0, sub1,
                st0, st1, st2, out_v, acc16, sem):
        wid = lax.axis_index("s") * NC + lax.axis_index("c")
        base = wid * b_per_w
        pltpu.sync_copy(idx_h.at[pl.ds(base, b_per_w)], idxbuf)
        lane_iota = lax.iota(jnp.int32, L)
        for f, blkf, subf in ((0, blk0, sub0), (1, blk1, sub1),
                              (2, blk2, None)):
            col = jnp.full((L,), f, jnp.int32)
            for g in range(b_per_w // L):
                vec = plsc.load_gather(idxbuf, [g * L + lane_iota, col])
                c, o = divmod(g * L, CH)
                if subf is None:
                    blkf[c, pl.ds(o, L)] = vec
                else:
                    blkf[c, pl.ds(o, L)] = vec >> 3
                    subf[c, pl.ds(o, L)] = vec & 7

        def chunk(c, carry):
            for grp in range(CH // L):
                sl16 = pl.ds(grp * L, L)
                b0 = blk0[c, sl16]
                s0 = sub0[c, sl16]
                b1 = blk1[c, sl16]
                s1 = sub1[c, sl16]
                b2 = blk2[c, sl16]
                for lb in range(L):
                    e = grp * L + lb
                    pltpu.async_copy(f0.at[b0[lb], s0[lb]], st0.at[e], sem)
                    pltpu.async_copy(f1.at[b1[lb], s1[lb]], st1.at[e], sem)
                    pltpu.async_copy(f2.at[b2[lb]], st2.at[e], sem)
            for stf in (st0, st1, st2):
                pltpu.make_async_copy(f2.at[pl.ds(0, CH)], stf, sem).wait()
            for grp in range(CH // L):
                for lb in range(L):
                    e = grp * L + lb
                    acc = (st0[e, pl.ds(0, L)] * st1[e, pl.ds(0, L)]
                           * st2[e, pl.ds(0, L)])
                    for j in range(1, n_k):
                        sl = pl.ds(j * L, L)
                        acc = acc + st0[e, sl] * st1[e, sl] * st2[e, sl]
                    acc16[lb, :] = acc
                tot = plsc.load_gather(
                    acc16, [lane_iota, jnp.zeros((L,), jnp.int32)])
                for col2 in range(1, L):
                    tot = tot + plsc.load_gather(
                        acc16, [lane_iota, jnp.full((L,), col2, jnp.int32)])
                out_v[pl.ds(c * CH + grp * L, L)] = tot
            return carry

        lax.fori_loop(0, n_chunks, chunk, 0)
        pltpu.sync_copy(out_v, out.at[pl.ds(base, b_per_w)])

    return parafac


def kernel(indices, F0, F1, F2):
    B = indices.shape[0]
    K = F0.shape[1]
    f0 = F0.reshape(-1, 8, K)
    f1 = F1.reshape(-1, 8, K)
    return _build(B, K)(indices.astype(jnp.int32), f0, f1, F2)


# speedup vs baseline: 1.0713x; 1.0713x over previous
import functools

import jax
import jax.numpy as jnp
from jax import lax
from jax.experimental import pallas as pl
from jax.experimental.pallas import tpu as pltpu
from jax.experimental.pallas import tpu_sc as plsc

NC = 2
NS = 16
NW = NC * NS
L = 16
CH = 32


@functools.lru_cache(maxsize=None)
def _build(B, K):
    assert B % (8 * NW) == 0
    b_per_w = B // NW
    n_chunks = b_per_w // CH
    n_k = K // L
    mesh = plsc.VectorSubcoreMesh(core_axis_name="c", subcore_axis_name="s")

    @functools.partial(
        pl.kernel,
        out_type=jax.ShapeDtypeStruct((B,), jnp.float32),
        mesh=mesh,
        compiler_params=pltpu.CompilerParams(needs_layout_passes=False),
        scratch_types=[
            pltpu.VMEM((b_per_w, 3), jnp.int32),
            pltpu.VMEM((n_chunks, CH), jnp.int32),
            pltpu.VMEM((n_chunks, CH), jnp.int32),
            pltpu.VMEM((n_chunks, CH), jnp.int32),
            pltpu.VMEM((n_chunks, CH), jnp.int32),
            pltpu.VMEM((n_chunks, CH), jnp.int32),
            pltpu.VMEM((CH, K), jnp.float32),
            pltpu.VMEM((CH, K), jnp.float32),
            pltpu.VMEM((CH, K), jnp.float32),
            pltpu.VMEM((b_per_w,), jnp.float32),
            pltpu.VMEM((L, L), jnp.float32),
            pltpu.SemaphoreType.DMA,
        ],
    )
    def parafac(idx_h, f0, f1, f2, out,
                idxbuf, blk0, blk1, blk2, sub0, sub1,
                st0, st1, st2, out_v, acc16, sem):
        wid = lax.axis_index("s") * NC + lax.axis_index("c")
        base = wid * b_per_w
        pltpu.sync_copy(idx_h.at[pl.ds(base, b_per_w)], idxbuf)
        lane_iota = lax.iota(jnp.int32, L)
        for f, blkf, subf in ((0, blk0, sub0), (1, blk1, sub1),
                              (2, blk2, None)):
            col = jnp.full((L,), f, jnp.int32)
            for g in range(b_per_w // L):
                vec = plsc.load_gather(idxbuf, [g * L + lane_iota, col])
                c, o = divmod(g * L, CH)
                if subf is None:
                    blkf[c, pl.ds(o, L)] = vec
                else:
                    blkf[c, pl.ds(o, L)] = vec >> 3
                    subf[c, pl.ds(o, L)] = vec & 7

        def chunk(c, carry):
            for grp in range(CH // L):
                sl16 = pl.ds(grp * L, L)
                b0 = blk0[c, sl16]
                s0 = sub0[c, sl16]
                b1 = blk1[c, sl16]
                s1 = sub1[c, sl16]
                b2 = blk2[c, sl16]
                for lb in range(L):
                    e = grp * L + lb
                    pltpu.async_copy(f0.at[b0[lb], s0[lb]], st0.at[e], sem)
                    pltpu.async_copy(f1.at[b1[lb], s1[lb]], st1.at[e], sem)
                    pltpu.async_copy(f2.at[b2[lb]], st2.at[e], sem)
            for stf in (st0, st1, st2):
                pltpu.make_async_copy(f2.at[pl.ds(0, CH)], stf, sem).wait()
            for grp in range(CH // L):
                for lb in range(L):
                    e = grp * L + lb
                    acc = (st0[e, pl.ds(0, L)] * st1[e, pl.ds(0, L)]
                           * st2[e, pl.ds(0, L)])
                    for j in range(1, n_k):
                        sl = pl.ds(j * L, L)
                        acc = acc + st0[e, sl] * st1[e, sl] * st2[e, sl]
                    acc16[lb, :] = acc
                tot = plsc.load_gather(
                    acc16, [lane_iota, jnp.zeros((L,), jnp.int32)])
                for col2 in range(1, L):
                    tot = tot + plsc.load_gather(
                        acc16, [lane_iota, jnp.full((L,), col2, jnp.int32)])
                out_v[pl.ds(c * CH + grp * L, L)] = tot
            return carry

        lax.fori_loop(0, n_chunks, chunk, 0)
        pltpu.sync_copy(out_v, out.at[pl.ds(base, b_per_w)])

    return parafac


def kernel(indices, F0, F1, F2):
    B = indices.shape[0]
    K = F0.shape[1]
    f0 = F0.reshape(-1, 8, K)
    f1 = F1.reshape(-1, 8, K)
    return _build(B, K)(indices.astype(jnp.int32), f0, f1, F2)
